# baseline (speedup 1.0000x reference)
import functools

import jax
import jax.numpy as jnp
from jax import lax
from jax.experimental import pallas as pl
from jax.experimental.pallas import tpu as pltpu
from jax.experimental.pallas import tpu_sc as plsc

B, L = 16384, 50
N = B * L
D0, D1, D2, DT = 80, 32, 16, 16
DOUT = D0 + D1 + D2 + 2 * DT
DPAD = 128

NC, NS, LANES = 2, 16, 16
NW = NC * NS
BROWS_W = B // NW
SUP = 8
NSUP = BROWS_W // SUP
NV = (L + LANES - 1) // LANES
TOFF0 = D0 + D1 + D2
TOFF1 = TOFF0 + DT

_mesh = plsc.VectorSubcoreMesh(
    core_axis_name="c", subcore_axis_name="s", num_cores=NC, num_subcores=NS
)

_scratch = (
    [pltpu.VMEM((SUP, L), jnp.int32)] * 10
    + [pltpu.VMEM((L, DPAD), jnp.float32)] * 4
    + [pltpu.VMEM((24, DT), jnp.float32)]
    + [pltpu.VMEM((7, DT), jnp.float32)]
    + [pltpu.VMEM((L, DOUT), jnp.float32)] * 4
    + [pltpu.SemaphoreType.DMA] * 8
)


@functools.partial(
    pl.kernel,
    mesh=_mesh,
    compiler_params=pltpu.CompilerParams(needs_layout_passes=False),
    out_type=jax.ShapeDtypeStruct((B, L, DOUT), jnp.float32),
    scratch_types=_scratch,
)
def _emb_kernel(x0_hbm, x1_hbm, x2_hbm, t0_hbm, t1_hbm,
                l0, l1, l2, tt0, tt1, out_hbm,
                xs00, xs10, xs20, ts00, ts10,
                xs01, xs11, xs21, ts01, ts11,
                b10, b20, b11, b21,
                t0v, t1v, cat0, cat1, cat2, cat3,
                ssem0, ssem1, gsem0, gsem1,
                wsem0, wsem1, wsem2, wsem3):
    xcols = [x0_hbm, x1_hbm, x2_hbm, t0_hbm, t1_hbm]
    stage = [[xs00, xs10, xs20, ts00, ts10], [xs01, xs11, xs21, ts01, ts11]]
    bufs = [[b10, b20], [b11, b21]]
    cat = [cat0, cat1, cat2, cat3]
    ssem, gsem = [ssem0, ssem1], [gsem0, gsem1]
    wsem = [wsem0, wsem1, wsem2, wsem3]

    wid = lax.axis_index("s") * NC + lax.axis_index("c")
    wrow = wid * BROWS_W

    pltpu.sync_copy(tt0, t0v)
    pltpu.sync_copy(tt1, t1v)

    def fire_staging(sph, srow):
        for m in range(5):
            pltpu.async_copy(xcols[m].at[pl.ds(srow, SUP)], stage[sph][m],
                             ssem[sph])

    def wait_staging(sph, srow):
        for m in range(5):
            pltpu.make_async_copy(xcols[m].at[pl.ds(srow, SUP)], stage[sph][m],
                                  ssem[sph]).wait()

    def gather_trips(sph, k, ph, r):
        yield l0.at[stage[sph][0].at[k]], cat[r].at[:, pl.ds(0, DPAD)], gsem[ph]
        yield l1.at[stage[sph][1].at[k]], bufs[ph][0], gsem[ph]
        yield l2.at[stage[sph][2].at[k]], bufs[ph][1], gsem[ph]

    def fire_gathers(sph, k, ph, r):
        for src, dst, sem in gather_trips(sph, k, ph, r):
            pltpu.async_copy(src, dst, sem)

    def wait_gathers(sph, k, ph, r):
        for src, dst, sem in gather_trips(sph, k, ph, r):
            pltpu.make_async_copy(src, dst, sem).wait()

    def time_assemble(sph, k, r):
        kv = jnp.full((LANES,), k, jnp.int32)

        def step(i, carry):
            p = jnp.minimum(lax.iota(jnp.int32, LANES) + i * LANES, L - 1)
            t0 = plsc.load_gather(stage[sph][3], [kv, p])
            t1 = plsc.load_gather(stage[sph][4], [kv, p])
            for j in range(DT):
                jv = jnp.full((LANES,), j, jnp.int32)
                v0 = plsc.load_gather(t0v, [t0, jv])
                plsc.store_scatter(cat[r], [p, jv + TOFF0], v0)
                v1 = plsc.load_gather(t1v, [t1, jv])
                plsc.store_scatter(cat[r], [p, jv + TOFF1], v1)
            return carry

        lax.fori_loop(0, NV, step, 0)

    def loc_assemble(ph, r):
        b1, b2 = bufs[ph]
        cr = cat[r]

        def step(rr, carry):
            for u in range(2):
                row = rr * 2 + u
                for j in range(D1 // LANES):
                    cr[row, pl.ds(D0 + j * LANES, LANES)] = (
                        b1[row, pl.ds(j * LANES, LANES)])
                cr[row, pl.ds(D0 + D1, LANES)] = b2[row, pl.ds(0, LANES)]
            return carry

        lax.fori_loop(0, L // 2, step, 0)

    fire_staging(0, wrow)
    wait_staging(0, wrow)
    fire_gathers(0, 0, 0, 0)

    def super_body(s, sph, carry):
        nsph = 1 - sph
        srow_n = wrow + jnp.minimum((s + 1) * SUP, BROWS_W - SUP)
        fire_staging(nsph, srow_n)

        def k_body(k):
            cc = s * SUP + k
            P, R, RN = k % 2, k % 4, (k + 1) % 4
            brow = wrow + cc

            @pl.when(cc >= 3)
            def _():
                pltpu.make_async_copy(cat[RN], out_hbm.at[brow], wsem[RN]).wait()

            if k == 6:
                wait_staging(nsph, srow_n)

            if k < 7:
                fire_gathers(sph, k + 1, (k + 1) % 2, (k + 1) % 4)
            else:
                fire_gathers(nsph, 0, 0, 0)

            time_assemble(sph, k, R)
            wait_gathers(sph, k, P, R)
            loc_assemble(P, R)
            pltpu.async_copy(cat[R], out_hbm.at[brow], wsem[R])

        for k in range(SUP):
            k_body(k)
        return carry

    def super_pair(i, carry):
        carry = super_body(i * 2, 0, carry)
        carry = super_body(i * 2 + 1, 1, carry)
        return carry

    lax.fori_loop(0, NSUP // 2, super_pair, 0)

    wait_gathers(0, 0, 0, 0)
    for r, dly in ((1, 3), (2, 2), (3, 1)):
        pltpu.make_async_copy(
            cat[r], out_hbm.at[wrow + BROWS_W - dly], wsem[r]).wait()


def kernel(x, t, loc_table0, loc_table1, loc_table2, time_table0, time_table1):
    l0p = jnp.pad(loc_table0, ((0, 0), (0, DPAD - D0)))
    l1p = jnp.pad(loc_table1, ((0, 0), (0, DPAD - D1)))
    l2p = jnp.pad(loc_table2, ((0, 0), (0, DPAD - D2)))
    return _emb_kernel(
        x[:, :, 0], x[:, :, 1], x[:, :, 2], t[:, :, 0], t[:, :, 1],
        l0p, l1p, l2p, time_table0, time_table1,
    )
